# Initial kernel scaffold; baseline (speedup 1.0000x reference)
import jax
import jax.numpy as jnp
from jax.experimental import pallas as pl


def kernel(x, mask):
    raise NotImplementedError("write your pallas kernel here")


# speedup vs baseline: 1.2468x; 1.2468x over previous
import jax
import jax.numpy as jnp
from jax import lax
from jax.experimental import pallas as pl
from jax.experimental.pallas import tpu as pltpu
from jax.experimental.pallas import tpu_sc as plsc

_R, _N = 128, 32768
_L = 16
_NC, _NS = 2, 16
_NW = _NC * _NS
_RPW = _R // _NW


def _sc_body(x_hbm, m_hbm, o_hbm, xv, mv, ov):
    wid = lax.axis_index("s") * _NC + lax.axis_index("c")

    def do_row(r, _):
        row = wid * _RPW + r
        pltpu.sync_copy(x_hbm.at[row], xv)
        pltpu.sync_copy(m_hbm.at[row], mv)

        def step(i, carry):
            off = i * _L
            v = xv[pl.ds(off, _L)] * mv[pl.ds(off, _L)]
            s = jnp.cumsum(v)
            ov[pl.ds(off, _L)] = s + carry
            return carry + jnp.sum(v)

        lax.fori_loop(0, _N // _L, step, jnp.float32(0.0), unroll=4)
        pltpu.sync_copy(ov, o_hbm.at[row])
        return 0

    lax.fori_loop(0, _RPW, do_row, 0)


def kernel(x, mask):
    maskf = mask.astype(jnp.float32)
    f = pl.kernel(
        _sc_body,
        out_type=jax.ShapeDtypeStruct((_R, _N), jnp.float32),
        mesh=plsc.VectorSubcoreMesh(core_axis_name="c", subcore_axis_name="s"),
        scratch_types=[
            pltpu.VMEM((_N,), jnp.float32),
            pltpu.VMEM((_N,), jnp.float32),
            pltpu.VMEM((_N,), jnp.float32),
        ],
        compiler_params=pltpu.CompilerParams(needs_layout_passes=False),
    )
    return f(x, maskf)
